# Initial kernel scaffold; baseline (speedup 1.0000x reference)
import jax
import jax.numpy as jnp
from jax.experimental import pallas as pl


def kernel(font_tokens, token_table, coord_x_table, coord_y_table, pos_table):
    raise NotImplementedError("write your pallas kernel here")


# speedup vs baseline: 4.0473x; 4.0473x over previous
import functools

import jax
import jax.numpy as jnp
from jax import lax
from jax.experimental import pallas as pl
from jax.experimental.pallas import tpu as pltpu
from jax.experimental.pallas import tpu_sc as plsc

D_MODEL = 512
FONT_X = 128
FONT_Y = 128
VOCAB = 512
BATCH = 32
SEQ = 2048

NUM_CORES = 2
NUM_SUBCORES = 16
NUM_WORKERS = NUM_CORES * NUM_SUBCORES
LANES = 16

S_CHUNK = SEQ // NUM_WORKERS


def _fuse_body(tok_ref, cxm_ref, cym_ref, cx1_ref, cy1_ref, out_ref):
    out_ref[0:FONT_X, :] = tok_ref[0:FONT_X, :] + cxm_ref[:, :] + cy1_ref[:, :]
    out_ref[FONT_X:FONT_X + FONT_Y, :] = (
        tok_ref[FONT_X:FONT_X + FONT_Y, :] + cx1_ref[:, :] + cym_ref[:, :])
    out_ref[FONT_X + FONT_Y:VOCAB, :] = (
        tok_ref[FONT_X + FONT_Y:VOCAB, :] + cx1_ref[:, :] + cy1_ref[:, :])


def _build_fused(token_table, coord_x_table, coord_y_table):
    cxm = coord_x_table[1:FONT_X + 1]
    cym = coord_y_table[1:FONT_Y + 1]
    cx1 = coord_x_table[1:2]
    cy1 = coord_y_table[1:2]
    return pl.pallas_call(
        _fuse_body,
        out_shape=jax.ShapeDtypeStruct((VOCAB, D_MODEL), jnp.float32),
    )(token_table, cxm, cym, cx1, cy1)


def _lookup_body(tok_hbm, fused_hbm, pos_hbm, out_hbm, idx_v, rows_v, pos_v, sem):
    wid = lax.axis_index("s") * NUM_CORES + lax.axis_index("c")
    s0 = wid * S_CHUNK
    pltpu.sync_copy(pos_hbm.at[pl.ds(s0, S_CHUNK)], pos_v)

    def batch_body(b, carry):
        pltpu.sync_copy(tok_hbm.at[pl.ds(b * SEQ + s0, S_CHUNK)], idx_v)
        pltpu.async_copy(fused_hbm.at[idx_v], rows_v, sem).wait()

        def row_body(j, c2):
            for k in range(D_MODEL // LANES):
                sl = pl.ds(k * LANES, LANES)
                rows_v[j, sl] = rows_v[j, sl] + pos_v[j, sl]
            return c2

        lax.fori_loop(0, S_CHUNK, row_body, 0)
        pltpu.sync_copy(rows_v, out_hbm.at[pl.ds(b * SEQ + s0, S_CHUNK)])
        return carry

    lax.fori_loop(0, BATCH, batch_body, 0)


_lookup = functools.partial(
    pl.kernel,
    out_type=jax.ShapeDtypeStruct((BATCH * SEQ, D_MODEL), jnp.float32),
    mesh=plsc.VectorSubcoreMesh(
        core_axis_name="c", subcore_axis_name="s",
        num_cores=NUM_CORES, num_subcores=NUM_SUBCORES),
    scratch_types=[
        pltpu.VMEM((S_CHUNK,), jnp.int32),
        pltpu.VMEM((S_CHUNK, D_MODEL), jnp.float32),
        pltpu.VMEM((S_CHUNK, D_MODEL), jnp.float32),
        pltpu.SemaphoreType.DMA,
    ],
)(_lookup_body)


def kernel(font_tokens, token_table, coord_x_table, coord_y_table, pos_table):
    fused = _build_fused(token_table, coord_x_table, coord_y_table)
    tokens_flat = font_tokens.astype(jnp.int32).reshape(BATCH * SEQ)
    out = _lookup(tokens_flat, fused, pos_table)
    return out.reshape(BATCH, SEQ, D_MODEL)


# speedup vs baseline: 5.7672x; 1.4249x over previous
import functools

import jax
import jax.numpy as jnp
from jax import lax
from jax.experimental import pallas as pl
from jax.experimental.pallas import tpu as pltpu
from jax.experimental.pallas import tpu_sc as plsc

D_MODEL = 512
FONT_X = 128
FONT_Y = 128
VOCAB = 512
BATCH = 32
SEQ = 2048

NUM_CORES = 2
NUM_SUBCORES = 16
NUM_WORKERS = NUM_CORES * NUM_SUBCORES
LANES = 16

S_OWN = SEQ // NUM_WORKERS
ROWS = 32
NBUF = 4
STEPS = BATCH * S_OWN // ROWS


def _fuse_body(tok_ref, cxm_ref, cym_ref, cx1_ref, cy1_ref, out_ref):
    out_ref[0:FONT_X, :] = tok_ref[0:FONT_X, :] + cxm_ref[:, :] + cy1_ref[:, :]
    out_ref[FONT_X:FONT_X + FONT_Y, :] = (
        tok_ref[FONT_X:FONT_X + FONT_Y, :] + cx1_ref[:, :] + cym_ref[:, :])
    out_ref[FONT_X + FONT_Y:VOCAB, :] = (
        tok_ref[FONT_X + FONT_Y:VOCAB, :] + cx1_ref[:, :] + cy1_ref[:, :])


def _build_fused(token_table, coord_x_table, coord_y_table):
    cxm = coord_x_table[1:FONT_X + 1]
    cym = coord_y_table[1:FONT_Y + 1]
    cx1 = coord_x_table[1:2]
    cy1 = coord_y_table[1:2]
    return pl.pallas_call(
        _fuse_body,
        out_shape=jax.ShapeDtypeStruct((VOCAB, D_MODEL), jnp.float32),
    )(token_table, cxm, cym, cx1, cy1)


def _lookup_body(tok_hbm, fused_hbm, pos_hbm, out_hbm,
                 idx_all, pos_v, rbufs, gsems, wsems, isem):
    wid = lax.axis_index("s") * NUM_CORES + lax.axis_index("c")
    s0 = wid * S_OWN
    pltpu.sync_copy(pos_hbm.at[pl.ds(s0, S_OWN)], pos_v)
    icps = [pltpu.async_copy(tok_hbm.at[pl.ds(b * SEQ + s0, S_OWN)],
                             idx_all.at[b], isem)
            for b in range(BATCH)]
    for c in icps:
        c.wait()

    def gather_start(t, p):
        idx_ref = idx_all.at[t // 2, pl.ds((t % 2) * ROWS, ROWS)]
        pltpu.async_copy(fused_hbm.at[idx_ref], rbufs[p], gsems[p])

    def wait_gather(p):
        pltpu.make_async_copy(
            fused_hbm.at[idx_all.at[0, pl.ds(0, ROWS)]],
            rbufs[p], gsems[p]).wait()

    def write_start(t, p):
        off = (t // 2) * SEQ + s0 + (t % 2) * ROWS
        pltpu.async_copy(rbufs[p], out_hbm.at[pl.ds(off, ROWS)], wsems[p])

    def wait_write(p):
        pltpu.make_async_copy(
            rbufs[p], out_hbm.at[pl.ds(0, ROWS)], wsems[p]).wait()

    gather_start(0, 0)

    def outer_body(g, carry):
        for ph in range(NBUF):
            t = g * NBUF + ph
            p1 = (ph + 1) % NBUF
            if ph < NBUF - 1:
                @pl.when(g >= 1)
                def _():
                    wait_write(p1)
                gather_start(t + 1, p1)
            else:
                wait_write(p1)

                @pl.when(g < STEPS // NBUF - 1)
                def _():
                    gather_start(t + 1, p1)
            wait_gather(ph)
            half = ph % 2

            def row_body(j, c2, _ph=ph, _half=half):
                for k in range(D_MODEL // LANES):
                    sl = pl.ds(k * LANES, LANES)
                    plsc.addupdate(rbufs[_ph].at[j, sl],
                                   pos_v[_half * ROWS + j, sl])
                return c2

            lax.fori_loop(0, ROWS, row_body, 0)
            write_start(t, ph)
        return carry

    lax.fori_loop(0, STEPS // NBUF, outer_body, 0)
    for p in range(1, NBUF):
        wait_write(p)


_lookup = functools.partial(
    pl.kernel,
    out_type=jax.ShapeDtypeStruct((BATCH * SEQ, D_MODEL), jnp.float32),
    mesh=plsc.VectorSubcoreMesh(
        core_axis_name="c", subcore_axis_name="s",
        num_cores=NUM_CORES, num_subcores=NUM_SUBCORES),
    scratch_types=[
        pltpu.VMEM((BATCH, S_OWN), jnp.int32),
        pltpu.VMEM((S_OWN, D_MODEL), jnp.float32),
        [pltpu.VMEM((ROWS, D_MODEL), jnp.float32) for _ in range(NBUF)],
        [pltpu.SemaphoreType.DMA for _ in range(NBUF)],
        [pltpu.SemaphoreType.DMA for _ in range(NBUF)],
        pltpu.SemaphoreType.DMA,
    ],
)(_lookup_body)


def kernel(font_tokens, token_table, coord_x_table, coord_y_table, pos_table):
    fused = _build_fused(token_table, coord_x_table, coord_y_table)
    tokens = font_tokens.astype(jnp.int32).reshape(BATCH * SEQ)
    out = _lookup(tokens, fused, pos_table)
    return out.reshape(BATCH, SEQ, D_MODEL)
